# Initial kernel scaffold; baseline (speedup 1.0000x reference)
import jax
import jax.numpy as jnp
from jax.experimental import pallas as pl


def kernel(x):
    raise NotImplementedError("write your pallas kernel here")


# speedup vs baseline: 9.3376x; 9.3376x over previous
import jax
import jax.numpy as jnp
from jax.experimental import pallas as pl

_K = 50
_ROWS_PER_BLOCK = 8


def _tc_body(x_ref, o_ref):
    x = x_ref[...]
    bi = jax.lax.bitcast_convert_type(x, jnp.int32)
    neg = bi >> 31
    keys = jax.lax.bitcast_convert_type(
        bi ^ (neg | jnp.int32(-2147483648)), jnp.uint32
    )

    def step(i, prefix):
        bit = jax.lax.shift_left(jnp.uint32(1), (31 - i).astype(jnp.uint32))
        cand = prefix | bit
        cnt = jnp.sum((keys >= cand).astype(jnp.int32), axis=1, keepdims=True)
        return jnp.where(cnt >= _K, cand, prefix)

    thr = jax.lax.fori_loop(0, 32, step, jnp.zeros((x.shape[0], 1), jnp.uint32))
    tb = jnp.where(thr >= jnp.uint32(0x80000000),
                   thr ^ jnp.uint32(0x80000000),
                   ~thr)
    tf = jax.lax.bitcast_convert_type(tb, jnp.float32)
    o_ref[...] = jnp.where(x < tf, -jnp.inf, x)


def kernel(x):
    b, n = x.shape
    grid = (b // _ROWS_PER_BLOCK,)
    return pl.pallas_call(
        _tc_body,
        grid=grid,
        in_specs=[pl.BlockSpec((_ROWS_PER_BLOCK, n), lambda i: (i, 0))],
        out_specs=pl.BlockSpec((_ROWS_PER_BLOCK, n), lambda i: (i, 0)),
        out_shape=jax.ShapeDtypeStruct((b, n), jnp.float32),
    )(x)
